# Initial kernel scaffold; baseline (speedup 1.0000x reference)
import jax
import jax.numpy as jnp
from jax.experimental import pallas as pl


def kernel(X, token_embedding, position_embedding):
    raise NotImplementedError("write your pallas kernel here")


# speedup vs baseline: 1.5126x; 1.5126x over previous
import functools

import jax
import jax.numpy as jnp
from jax import lax
from jax.experimental import pallas as pl
from jax.experimental.pallas import tpu as pltpu
from jax.experimental.pallas import tpu_sc as plsc

_ALPHABET = 1000
_SEQ = 50
_EMB = 1052
_BATCH = 1024

_NC, _NS = 2, 16
_NW = _NC * _NS
_NB = _BATCH // _NW
_FULL = _EMB // 16
_TAILOFF = _EMB - 16
_OVERLAP = _FULL * 16 - _TAILOFF

_mesh = plsc.VectorSubcoreMesh(
    core_axis_name="c", subcore_axis_name="s", num_cores=_NC, num_subcores=_NS
)


@functools.partial(
    pl.kernel,
    out_type=jax.ShapeDtypeStruct((_BATCH * _SEQ, _EMB), jnp.float32),
    mesh=_mesh,
    compiler_params=pltpu.CompilerParams(use_tc_tiling_on_sc=False),
    scratch_types=[
        pltpu.VMEM((_SEQ, _EMB), jnp.float32),
        pltpu.VMEM((_SEQ, _NB), jnp.int32),
        pltpu.VMEM((_SEQ, _NB), jnp.int32),
        pltpu.VMEM((_NB, _EMB), jnp.float32),
        pltpu.VMEM((_NB, _EMB), jnp.float32),
        pltpu.SemaphoreType.DMA,
        pltpu.SemaphoreType.DMA,
        pltpu.SemaphoreType.DMA,
        pltpu.SemaphoreType.DMA,
    ],
)
def _emb_kernel(xtw_hbm, oidw_hbm, tok_hbm, pos_hbm, out_hbm,
                posv, xtv, oidv, buf0, buf1, gs0, gs1, ws0, ws1):
    wid = lax.axis_index("s") * _NC + lax.axis_index("c")
    bufs = (buf0, buf1)
    gsems = (gs0, gs1)
    wsems = (ws0, ws1)

    pltpu.sync_copy(pos_hbm, posv)
    pltpu.sync_copy(xtw_hbm.at[wid], xtv)
    pltpu.sync_copy(oidw_hbm.at[wid], oidv)

    def gather_start(s, p):
        pltpu.async_copy(tok_hbm.at[xtv.at[s]], bufs[p], gsems[p])

    def gather_wait(p):
        pltpu.make_async_copy(tok_hbm.at[xtv.at[0]], bufs[p], gsems[p]).wait()

    def write_start(s, p):
        pltpu.async_copy(bufs[p], out_hbm.at[oidv.at[s]], wsems[p])

    def write_wait(p):
        pltpu.make_async_copy(bufs[p], out_hbm.at[oidv.at[0]], wsems[p]).wait()

    lanes = lax.iota(jnp.int32, 16)
    tail_keep = lanes >= _OVERLAP

    def compute(s, p):
        buf = bufs[p]

        def jbody(j, carry):
            off = j * 16
            pj = posv[s, pl.ds(off, 16)]
            for i in range(_NB):
                plsc.addupdate(buf.at[i, pl.ds(off, 16)], pj)
            return carry

        lax.fori_loop(0, _FULL, jbody, 0)

        pt = jnp.where(tail_keep, posv[s, pl.ds(_TAILOFF, 16)], 0.0)
        for i in range(_NB):
            plsc.addupdate(buf.at[i, pl.ds(_TAILOFF, 16)], pt)

    gather_start(0, 0)
    gather_start(1, 1)

    def sbody(g, carry):
        for p in range(2):
            s = 2 * g + p
            gather_wait(p)
            compute(s, p)
            write_start(s, p)
        for p in range(2):
            s = 2 * g + p
            write_wait(p)
            gather_start(s + 2, p)
        return carry

    lax.fori_loop(0, _SEQ // 2 - 1, sbody, 0)

    for p in range(2):
        s = _SEQ - 2 + p
        gather_wait(p)
        compute(s, p)
        write_start(s, p)
    for p in range(2):
        write_wait(p)


def kernel(X, token_embedding, position_embedding):
    X = X.astype(jnp.int32)
    xtw = X.T.reshape(_SEQ, _NW, _NB).transpose(1, 0, 2)
    rows = jnp.arange(_BATCH, dtype=jnp.int32) * _SEQ
    oid = rows[:, None] + jnp.arange(_SEQ, dtype=jnp.int32)[None, :]
    oidw = oid.T.reshape(_SEQ, _NW, _NB).transpose(1, 0, 2)
    out_flat = _emb_kernel(xtw, oidw, token_embedding, position_embedding)
    return out_flat.reshape(_BATCH, _SEQ, _EMB)
